# Initial kernel scaffold; baseline (speedup 1.0000x reference)
import jax
import jax.numpy as jnp
from jax.experimental import pallas as pl


def kernel(x, edge_index, edge_attr, Wf1, bf1, Ws1, bs1, Wf2, bf2, Ws2, bs2, Wf3, bf3, Ws3, bs3):
    raise NotImplementedError("write your pallas kernel here")


# speedup vs baseline: 1.9017x; 1.9017x over previous
import functools

import jax
import jax.numpy as jnp
from jax import lax
from jax.experimental import pallas as pl
from jax.experimental.pallas import tpu as pltpu
from jax.experimental.pallas import tpu_sc as plsc

N = 10000
E = 320000
C = 128
D = 16
C2 = 2 * C
H = C // 2
HW = 2 * H

_LOG1P_C = (
    0.010485872356292735,
    -0.054175141702717526,
    0.1333511583532261,
    -0.22500693106106187,
    0.3279371811874688,
    -0.4994232971539362,
    0.9999785162766249,
)


_BE = 1280


def _ef_body(ea_ref, w_ref, *o_refs):
    out = jnp.dot(ea_ref[...], w_ref[...], preferred_element_type=jnp.float32)
    for i, o in enumerate(o_refs):
        o[...] = out[:, i * HW:(i + 1) * HW]


def _edge_proj(edge_attr, we_all):
    grid = E // _BE
    return pl.pallas_call(
        _ef_body,
        grid=(grid,),
        in_specs=[
            pl.BlockSpec((_BE, D), lambda i: (i, 0)),
            pl.BlockSpec((D, 6 * HW), lambda i: (0, 0)),
        ],
        out_specs=[pl.BlockSpec((_BE, HW), lambda i: (i, 0))] * 6,
        out_shape=[jax.ShapeDtypeStruct((E, HW), jnp.float32)] * 6,
    )(edge_attr, we_all)


_BN = 1000


def _proj_first_body(a_ref, wd_ref, ws_ref, bd_ref, td0, td1, ts0, ts1):
    h = a_ref[...]
    td = jnp.dot(h, wd_ref[...], preferred_element_type=jnp.float32) + bd_ref[...]
    ts = jnp.dot(h, ws_ref[...], preferred_element_type=jnp.float32)
    td0[...] = td[:, :HW]
    td1[...] = td[:, HW:]
    ts0[...] = ts[:, :HW]
    ts1[...] = ts[:, HW:]


def _proj_first(a, wd, ws, bd):
    grid = N // _BN
    return pl.pallas_call(
        _proj_first_body,
        grid=(grid,),
        in_specs=[
            pl.BlockSpec((_BN, C), lambda i: (i, 0)),
            pl.BlockSpec((C, C2), lambda i: (0, 0)),
            pl.BlockSpec((C, C2), lambda i: (0, 0)),
            pl.BlockSpec((1, C2), lambda i: (0, 0)),
        ],
        out_specs=[pl.BlockSpec((_BN, HW), lambda i: (i, 0))] * 4,
        out_shape=[jax.ShapeDtypeStruct((N, HW), jnp.float32)] * 4,
    )(a, wd, ws, bd)


def _proj_mid_body(a_ref, q00, q01, q10, q11, wd_ref, ws_ref, bd_ref,
                   h_ref, td0, td1, ts0, ts1):
    agg = jnp.concatenate([q00[...] + q01[...], q10[...] + q11[...]], axis=1)
    h = jnp.maximum(a_ref[...] + agg, 0.0)
    h_ref[...] = h
    td = jnp.dot(h, wd_ref[...], preferred_element_type=jnp.float32) + bd_ref[...]
    ts = jnp.dot(h, ws_ref[...], preferred_element_type=jnp.float32)
    td0[...] = td[:, :HW]
    td1[...] = td[:, HW:]
    ts0[...] = ts[:, :HW]
    ts1[...] = ts[:, HW:]


def _proj_mid(a, q00, q01, q10, q11, wd, ws, bd):
    grid = N // _BN
    return pl.pallas_call(
        _proj_mid_body,
        grid=(grid,),
        in_specs=[
            pl.BlockSpec((_BN, C), lambda i: (i, 0)),
            pl.BlockSpec((_BN, H), lambda i: (i, 0)),
            pl.BlockSpec((_BN, H), lambda i: (i, 0)),
            pl.BlockSpec((_BN, H), lambda i: (i, 0)),
            pl.BlockSpec((_BN, H), lambda i: (i, 0)),
            pl.BlockSpec((C, C2), lambda i: (0, 0)),
            pl.BlockSpec((C, C2), lambda i: (0, 0)),
            pl.BlockSpec((1, C2), lambda i: (0, 0)),
        ],
        out_specs=[pl.BlockSpec((_BN, C), lambda i: (i, 0))]
        + [pl.BlockSpec((_BN, HW), lambda i: (i, 0))] * 4,
        out_shape=[jax.ShapeDtypeStruct((N, C), jnp.float32)]
        + [jax.ShapeDtypeStruct((N, HW), jnp.float32)] * 4,
    )(a, q00, q01, q10, q11, wd, ws, bd)


def _final_body(a_ref, q00, q01, q10, q11, o_ref):
    agg = jnp.concatenate([q00[...] + q01[...], q10[...] + q11[...]], axis=1)
    h = a_ref[...] + agg
    m = jnp.max(h, axis=1, keepdims=True)
    e = jnp.exp(h - m)
    o_ref[...] = e / jnp.sum(e, axis=1, keepdims=True)


def _final(a, q00, q01, q10, q11):
    grid = N // _BN
    return pl.pallas_call(
        _final_body,
        grid=(grid,),
        in_specs=[
            pl.BlockSpec((_BN, C), lambda i: (i, 0)),
            pl.BlockSpec((_BN, H), lambda i: (i, 0)),
            pl.BlockSpec((_BN, H), lambda i: (i, 0)),
            pl.BlockSpec((_BN, H), lambda i: (i, 0)),
            pl.BlockSpec((_BN, H), lambda i: (i, 0)),
        ],
        out_specs=pl.BlockSpec((_BN, C), lambda i: (i, 0)),
        out_shape=jax.ShapeDtypeStruct((N, C), jnp.float32),
    )(a, q00, q01, q10, q11)


_CHUNK = 80
_NC = 2
_NS = 16
_EPW = E // (_NC * _NS)
_NIT = _EPW // _CHUNK
_AROWS = 5120
_RPT = _AROWS // _NS
_ZCH = 80


def _sc_edge_body(tdst_hbm, tsrc_hbm, ef_hbm, dst_hbm, src_hbm, out_hbm,
                  idx_d, idx_s, idx2, rows_d, rows_s, ef_v, msg_v, zbuf,
                  accum, sem1, sem2):
    c = lax.axis_index("c")
    s = lax.axis_index("s")
    wid = c * _NS + s

    zero16 = jnp.zeros((16,), jnp.float32)

    def zrow(i, carry):
        for j in range(HW // 16):
            zbuf[i, pl.ds(j * 16, 16)] = zero16
        return carry

    lax.fori_loop(0, _ZCH, zrow, 0)

    r0 = s * _RPT

    def zacc(i, carry):
        pltpu.sync_copy(zbuf, accum.at[pl.ds(r0 + i * _ZCH, _ZCH)])
        return carry

    lax.fori_loop(0, _RPT // _ZCH, zacc, 0)
    plsc.subcore_barrier()

    base_w = wid * _EPW

    def step(it, carry):
        base = base_w + it * _CHUNK
        pltpu.sync_copy(dst_hbm.at[pl.ds(base, _CHUNK)], idx_d.at[pl.ds(0, _CHUNK)])
        pltpu.sync_copy(src_hbm.at[pl.ds(base, _CHUNK)], idx_s)
        cp1 = pltpu.async_copy(tdst_hbm.at[idx_d.at[pl.ds(0, _CHUNK)]], rows_d, sem1)
        cp2 = pltpu.async_copy(tsrc_hbm.at[idx_s], rows_s, sem2)
        pltpu.sync_copy(ef_hbm.at[pl.ds(base, _CHUNK)], ef_v)
        for k in range(_CHUNK // 16):
            idx2[pl.ds(k * 16, 16)] = lax.shift_right_logical(
                idx_d[pl.ds(k * 16, 16)], 1)
        cp1.wait()
        cp2.wait()

        def edge(e, ecarry):
            dvec = idx_d[pl.ds(e, 16)]
            pv = (dvec[0] & 1).astype(jnp.float32)
            pvec = jnp.full((16,), pv, jnp.float32)
            qvec = 1.0 - pvec
            for j in range(H // 16):
                lo = j * 16
                hi = H + j * 16
                af = rows_d[e, pl.ds(lo, 16)] + rows_s[e, pl.ds(lo, 16)] \
                    + ef_v[e, pl.ds(lo, 16)]
                av = rows_d[e, pl.ds(hi, 16)] + rows_s[e, pl.ds(hi, 16)] \
                    + ef_v[e, pl.ds(hi, 16)]
                af = jnp.maximum(af, -30.0)
                f = 1.0 / (1.0 + jnp.exp(-af))
                m = jnp.maximum(av, 0.0)
                u = jnp.exp(-jnp.abs(av))
                p = _LOG1P_C[0]
                for coef in _LOG1P_C[1:]:
                    p = p * u + coef
                p = p * u
                msg = f * (m + p)
                msg_v[e, pl.ds(lo, 16)] = msg * qvec
                msg_v[e, pl.ds(H + lo, 16)] = msg * pvec
            return ecarry

        lax.fori_loop(0, _CHUNK, edge, 0)
        pltpu.sync_copy(msg_v, accum.at[idx2], add=True)
        return carry

    lax.fori_loop(0, _NIT, step, 0)
    plsc.subcore_barrier()
    pltpu.sync_copy(accum.at[pl.ds(r0, _RPT)], out_hbm.at[c, pl.ds(r0, _RPT)])


def _sc_edge(tdst, tsrc, ef, dst, src):
    mesh = plsc.VectorSubcoreMesh(core_axis_name="c", subcore_axis_name="s")
    return pl.kernel(
        _sc_edge_body,
        out_type=jax.ShapeDtypeStruct((_NC, _AROWS, HW), jnp.float32),
        mesh=mesh,
        scratch_types=[
            pltpu.VMEM((_CHUNK + 16,), jnp.int32),
            pltpu.VMEM((_CHUNK,), jnp.int32),
            pltpu.VMEM((_CHUNK,), jnp.int32),
            pltpu.VMEM((_CHUNK, HW), jnp.float32),
            pltpu.VMEM((_CHUNK, HW), jnp.float32),
            pltpu.VMEM((_CHUNK, HW), jnp.float32),
            pltpu.VMEM((_CHUNK, HW), jnp.float32),
            pltpu.VMEM((_ZCH, HW), jnp.float32),
            pltpu.VMEM_SHARED((_AROWS, HW), jnp.float32),
            pltpu.SemaphoreType.DMA,
            pltpu.SemaphoreType.DMA,
        ],
    )(tdst, tsrc, ef, dst, src)


def kernel(x, edge_index, edge_attr, Wf1, bf1, Ws1, bs1, Wf2, bf2, Ws2, bs2,
           Wf3, bf3, Ws3, bs3):
    src = edge_index[0].astype(jnp.int32)
    dst = edge_index[1].astype(jnp.int32)

    def parts(Wf, bf, Ws, bs):
        def perm(wf_t, ws_t):
            return jnp.concatenate([wf_t[:, :H], ws_t[:, :H],
                                    wf_t[:, H:], ws_t[:, H:]], axis=1)

        wd = perm(Wf[:, :C].T, Ws[:, :C].T)
        wsrc = perm(Wf[:, C:C2].T, Ws[:, C:C2].T)
        we = perm(Wf[:, C2:].T, Ws[:, C2:].T)
        bd = jnp.concatenate([bf[:H], bs[:H], bf[H:], bs[H:]]).reshape(1, C2)
        return wd, wsrc, we, bd

    wd1, ws1_, we1, bd1 = parts(Wf1, bf1, Ws1, bs1)
    wd2, ws2_, we2, bd2 = parts(Wf2, bf2, Ws2, bs2)
    wd3, ws3_, we3, bd3 = parts(Wf3, bf3, Ws3, bs3)

    efs = _edge_proj(edge_attr, jnp.concatenate([we1, we2, we3], axis=1))
    ef = {(1, 0): efs[0], (1, 1): efs[1], (2, 0): efs[2], (2, 1): efs[3],
          (3, 0): efs[4], (3, 1): efs[5]}

    dep = jnp.float32(0.0)

    def run_layer(tds, tss, k, dep):
        qs = []
        for hh in (0, 1):
            td, ts_, ef_h, _d = lax.optimization_barrier(
                (tds[hh], tss[hh], ef[(k, hh)], dep))
            agg = _sc_edge(td, ts_, ef_h, dst, src)
            dep = agg[0, 0, 0]
            qs.append(agg.reshape(_NC, 2 * _AROWS, H)[:, :N])
        return (qs[0][0], qs[0][1], qs[1][0], qs[1][1]), dep

    td0, td1, ts0, ts1 = _proj_first(x, wd1, ws1_, bd1)
    q, dep = run_layer((td0, td1), (ts0, ts1), 1, dep)
    a = x
    for wd, ws, bd, k in ((wd2, ws2_, bd2, 2), (wd2, ws2_, bd2, 2),
                          (wd3, ws3_, bd3, 3)):
        a, td0, td1, ts0, ts1 = _proj_mid(a, *q, wd, ws, bd)
        q, dep = run_layer((td0, td1), (ts0, ts1), k, dep)

    return _final(a, *q)
